# baseline (speedup 1.0000x reference)
import functools

import jax
import jax.numpy as jnp
from jax import lax
from jax.experimental import pallas as pl
from jax.experimental.pallas import tpu as pltpu

_BMA = 304
_BMO = 256


def _body(adj_ref, f_ref, wb_ref, wm_ref, wl_ref, noise_ref, o_ref,
          p_ref, g_ref, z_ref, *, n, nba, nbo, d_emb):
    i = pl.program_id(0)

    @pl.when(i == 0)
    def _phase_p():
        wcat = jnp.concatenate([wm_ref[...], wl_ref[...]], axis=1)
        wc = jnp.dot(wb_ref[...], wcat, preferred_element_type=jnp.float32)
        p_ref[...] = jnp.dot(f_ref[...], wc,
                             preferred_element_type=jnp.float32)

    @pl.when(i < nba)
    def _phase_g():
        r = i * _BMA
        g_ref[pl.ds(r, _BMA), :] = jnp.dot(
            adj_ref[...], p_ref[...],
            preferred_element_type=jnp.float32)

    @pl.when((i >= nba) & (i < 2 * nba))
    def _phase_z():
        r = (i - nba) * _BMA
        m = jnp.maximum(jnp.dot(adj_ref[...], g_ref[:n, :],
                                preferred_element_type=jnp.float32), 0.0)
        mean = m[:, :d_emb]
        logstd = m[:, d_emb:]
        z_ref[pl.ds(r, _BMA), :] = (
            noise_ref[...] * jnp.exp(logstd) + mean)

    @pl.when(i >= 2 * nba)
    def _phase_out():
        r = (i - 2 * nba) * _BMO
        zi = z_ref[pl.ds(r, _BMO), :]
        o_ref[...] = lax.dot_general(
            zi, z_ref[:n, :], (((1,), (1,)), ((), ())),
            preferred_element_type=jnp.float32)


def kernel(adj, features, W_base, W_mean, W_logstd, noise):
    n, d_in = features.shape
    d_hid = W_base.shape[1]
    d_emb = W_mean.shape[1]
    d2 = 2 * d_emb
    nba = -(-n // _BMA)
    nbo = -(-n // _BMO)
    npad = max(nba * _BMA, nbo * _BMO)

    def adj_map(i):
        return (jnp.where(i < nba, i,
                          jnp.where(i < 2 * nba, i - nba, nba - 1)), 0)

    def noise_map(i):
        return (jnp.clip(i - nba, 0, nba - 1), 0)

    def out_map(i):
        return (jnp.where(i >= 2 * nba, i - 2 * nba, 0), 0)

    body = functools.partial(_body, n=n, nba=nba, nbo=nbo, d_emb=d_emb)

    out = pl.pallas_call(
        body,
        grid=(2 * nba + nbo,),
        in_specs=[
            pl.BlockSpec((_BMA, n), adj_map),
            pl.BlockSpec((n, d_in), lambda i: (0, 0)),
            pl.BlockSpec((d_in, d_hid), lambda i: (0, 0)),
            pl.BlockSpec((d_hid, d_emb), lambda i: (0, 0)),
            pl.BlockSpec((d_hid, d_emb), lambda i: (0, 0)),
            pl.BlockSpec((_BMA, d_emb), noise_map),
        ],
        out_specs=pl.BlockSpec((_BMO, n), out_map),
        out_shape=jax.ShapeDtypeStruct((n, n), jnp.float32),
        scratch_shapes=[
            pltpu.VMEM((n, d2), jnp.float32),
            pltpu.VMEM((npad, d2), jnp.float32),
            pltpu.VMEM((npad, d_emb), jnp.float32),
        ],
        compiler_params=pltpu.CompilerParams(
            vmem_limit_bytes=100 * 1024 * 1024),
    )(adj, features, W_base, W_mean, W_logstd, noise)

    return out
